# baseline (speedup 1.0000x reference)
import functools

import jax
import jax.numpy as jnp
from jax import lax
from jax.experimental import pallas as pl
from jax.experimental.pallas import tpu as pltpu
from jax.experimental.pallas import tpu_sc as plsc

B, C, T, K = 64, 256, 2048, 32

L = 16
H = T // 2
NBUF = 6
AHEAD = 4
NUM_CHUNKS = 2 * B // L


def _selector_dup_kernel(x_hbm, alpha_hbm, z_hbm, w_hbm, w2_hbm,
                         arow_v, wrow_v,
                         buf0, buf1, buf2, buf3, buf4, buf5,
                         gsem0, gsem1, gsem2, gsem3, gsem4, gsem5,
                         ssem0, ssem1, ssem2, ssem3, ssem4, ssem5):
    nc = 2
    wid = lax.axis_index("s") * nc + lax.axis_index("c")

    pltpu.sync_copy(alpha_hbm.at[wid], arow_v)
    iota = lax.iota(jnp.int32, L)
    best_v = arow_v[pl.ds(0, L)]
    best_i = iota
    for j in range(1, C // L):
        v = arow_v[pl.ds(j * L, L)]
        pos = iota + j * L
        upd = v > best_v
        best_v = jnp.where(upd, v, best_v)
        best_i = jnp.where(upd, pos, best_i)
    sk, _ = plsc.sort_key_val(best_v, best_i)
    m = sk[15]
    cand = jnp.where(best_v == m, best_i, jnp.int32(C))
    ck_sorted, _ = plsc.sort_key_val(cand, cand)
    c_k = ck_sorted[0]

    bufs = (buf0, buf1, buf2, buf3, buf4, buf5)
    gsems = (gsem0, gsem1, gsem2, gsem3, gsem4, gsem5)
    ssems = (ssem0, ssem1, ssem2, ssem3, ssem4, ssem5)

    def gidx(ch):
        hr = iota + ch * L
        return (hr >> 1) * (2 * C) + 2 * c_k + (hr & 1)

    def sidx(ch):
        hr = iota + ch * L
        return (hr >> 1) * (2 * K) + 2 * wid + (hr & 1)

    def gather(ch):
        return pltpu.async_copy(x_hbm.at[gidx(ch)], bufs[ch % NBUF],
                                gsems[ch % NBUF])

    gathers = [None] * NUM_CHUNKS
    scatters = [None] * NUM_CHUNKS
    for ch in range(AHEAD):
        gathers[ch] = gather(ch)

    for j in range(C // L):
        pos = iota + j * L
        wrow_v[pl.ds(j * L, L)] = jnp.where(pos == c_k, 1.0, 0.0).astype(
            jnp.float32)
    pltpu.sync_copy(wrow_v, w_hbm.at[wid])
    pltpu.sync_copy(wrow_v, w2_hbm.at[wid])

    for ch in range(NUM_CHUNKS):
        nxt = ch + AHEAD
        if nxt < NUM_CHUNKS:
            if nxt - NBUF >= 0:
                scatters[nxt - NBUF].wait()
            gathers[nxt] = gather(nxt)
        gathers[ch].wait()
        scatters[ch] = pltpu.async_copy(
            bufs[ch % NBUF], z_hbm.at[sidx(ch)], ssems[ch % NBUF])
    for ch in range(max(0, NUM_CHUNKS - NBUF), NUM_CHUNKS):
        scatters[ch].wait()


@jax.jit
def _run(x_flat, alpha):
    mesh = plsc.VectorSubcoreMesh(core_axis_name="c", subcore_axis_name="s")
    fn = functools.partial(
        pl.kernel, mesh=mesh,
        compiler_params=pltpu.CompilerParams(needs_layout_passes=False),
        out_type=[
            jax.ShapeDtypeStruct((B * K * 2, H), jnp.float32),
            jax.ShapeDtypeStruct((K, C), jnp.float32),
            jax.ShapeDtypeStruct((K, C), jnp.float32),
        ],
        scratch_types=(
            [pltpu.VMEM((C,), jnp.float32)] * 2
            + [pltpu.VMEM((L, H), jnp.float32)] * NBUF
            + [pltpu.SemaphoreType.DMA] * (2 * NBUF)
        ),
    )(_selector_dup_kernel)
    return fn(x_flat, alpha)


def kernel(x, alpha):
    z_flat, w_hard, w_hard2 = _run(x.reshape(B * C * 2, H), alpha)
    return (z_flat.reshape(B, 1, K, T), w_hard, w_hard2)


# speedup vs baseline: 4.1276x; 4.1276x over previous
import functools

import jax
import jax.numpy as jnp
from jax import lax
from jax.experimental import pallas as pl
from jax.experimental.pallas import tpu as pltpu
from jax.experimental.pallas import tpu_sc as plsc

B, C, T, K = 64, 256, 2048, 32

L = 16
NBUF = 3
NUM_CHUNKS = 4


def _selector_dup_kernel(x_hbm, alphat_hbm, z_hbm, w_hbm, w2_hbm,
                         at_v, wrow_v,
                         buf0, buf1, buf2,
                         gsem0, gsem1, gsem2,
                         ssem0, ssem1, ssem2):
    nc = 2
    core = lax.axis_index("c")
    sid = lax.axis_index("s")
    wid = sid * nc + core
    iota = lax.iota(jnp.int32, L)

    def sweep(half):
        best = None
        ci = jnp.zeros((L,), jnp.int32)
        for q in range(2):
            pltpu.sync_copy(alphat_hbm.at[half, pl.ds(q * (C // 2), C // 2)],
                            at_v)
            for c in range(C // 2):
                v = at_v[c, :]
                if best is None:
                    best = v
                    continue
                upd = v > best
                best = jnp.where(upd, v, best)
                ci = jnp.where(upd,
                               jnp.full((L,), q * (C // 2) + c, jnp.int32), ci)
        return ci

    cA = sweep(0)
    cB = sweep(1)

    k0 = 2 * sid
    lane = k0 % L
    src = jnp.where(sid < 8, cA, cB)
    c0s, _ = plsc.sort_key_val(jnp.where(iota == lane, src, jnp.int32(C)),
                               iota)
    c1s, _ = plsc.sort_key_val(jnp.where(iota == lane + 1, src, jnp.int32(C)),
                               iota)
    for krow, ck in ((0, c0s[0]), (1, c1s[0])):
        for j in range(C // L):
            pos = iota + j * L
            wrow_v[pl.ds(j * L, L)] = jnp.where(pos == ck, 1.0, 0.0).astype(
                jnp.float32)

        @pl.when(core == 0)
        def _():
            pltpu.sync_copy(wrow_v, w_hbm.at[k0 + krow])
            pltpu.sync_copy(wrow_v, w2_hbm.at[k0 + krow])

    bufs = (buf0, buf1, buf2)
    gsems = (gsem0, gsem1, gsem2)
    ssems = (ssem0, ssem1, ssem2)

    def src_idx(ch):
        b = 2 * wid + (ch // 2)
        return b * C + (cA if ch % 2 == 0 else cB)

    def dst_base(ch):
        b = 2 * wid + (ch // 2)
        return b * K + (ch % 2) * L

    def gather(ch):
        return pltpu.async_copy(x_hbm.at[src_idx(ch)], bufs[ch % NBUF],
                                gsems[ch % NBUF])

    gathers = [None] * NUM_CHUNKS
    scatters = [None] * NUM_CHUNKS
    for ch in range(NBUF - 1):
        gathers[ch] = gather(ch)

    for ch in range(NUM_CHUNKS):
        nxt = ch + NBUF - 1
        if nxt < NUM_CHUNKS:
            if ch >= 1:
                scatters[ch - 1].wait()
            gathers[nxt] = gather(nxt)
        gathers[ch].wait()
        scatters[ch] = pltpu.async_copy(
            bufs[ch % NBUF], z_hbm.at[pl.ds(dst_base(ch), L)],
            ssems[ch % NBUF])
    for ch in range(max(0, NUM_CHUNKS - NBUF), NUM_CHUNKS):
        scatters[ch].wait()


@jax.jit
def _run(x_flat, alphat):
    mesh = plsc.VectorSubcoreMesh(core_axis_name="c", subcore_axis_name="s")
    fn = functools.partial(
        pl.kernel, mesh=mesh,
        compiler_params=pltpu.CompilerParams(needs_layout_passes=False),
        out_type=[
            jax.ShapeDtypeStruct((B * K, T), jnp.float32),
            jax.ShapeDtypeStruct((K, C), jnp.float32),
            jax.ShapeDtypeStruct((K, C), jnp.float32),
        ],
        scratch_types=(
            [pltpu.VMEM((C // 2, L), jnp.float32),
             pltpu.VMEM((C,), jnp.float32)]
            + [pltpu.VMEM((L, T), jnp.float32)] * NBUF
            + [pltpu.SemaphoreType.DMA] * (2 * NBUF)
        ),
    )(_selector_dup_kernel)
    return fn(x_flat, alphat)


def kernel(x, alpha):
    alphat = jnp.moveaxis(alpha.T.reshape(C, 2, L), 1, 0)
    z_flat, w_hard, w_hard2 = _run(x.reshape(B * C, T), alphat)
    return (z_flat.reshape(B, 1, K, T), w_hard, w_hard2)


# speedup vs baseline: 5.1100x; 1.2380x over previous
import functools

import jax
import jax.numpy as jnp
from jax import lax
from jax.experimental import pallas as pl
from jax.experimental.pallas import tpu as pltpu
from jax.experimental.pallas import tpu_sc as plsc

B, C, T, K = 64, 256, 2048, 32

L = 16
NBUF = 3
ROWS_PER_CHUNK = 16
NUM_CHUNKS = B // ROWS_PER_CHUNK


def _selector_dup_kernel(x_hbm, alpha_hbm, z_hbm, w_hbm, w2_hbm,
                         arow_v, wrow_v,
                         buf0, buf1, buf2,
                         gsem0, gsem1, gsem2,
                         ssem0, ssem1, ssem2):
    nc = 2
    wid = lax.axis_index("s") * nc + lax.axis_index("c")

    pltpu.sync_copy(alpha_hbm.at[wid], arow_v)
    iota = lax.iota(jnp.int32, L)
    best_v = arow_v[pl.ds(0, L)]
    best_i = iota
    for j in range(1, C // L):
        v = arow_v[pl.ds(j * L, L)]
        pos = iota + j * L
        upd = v > best_v
        best_v = jnp.where(upd, v, best_v)
        best_i = jnp.where(upd, pos, best_i)
    sk, _ = plsc.sort_key_val(best_v, best_i)
    m = sk[15]
    cand = jnp.where(best_v == m, best_i, jnp.int32(C))
    ck_sorted, _ = plsc.sort_key_val(cand, cand)
    c_k = ck_sorted[0]

    bufs = (buf0, buf1, buf2)
    gsems = (gsem0, gsem1, gsem2)
    ssems = (ssem0, ssem1, ssem2)

    def gidx(ch):
        return (iota + ch * ROWS_PER_CHUNK) * C + c_k

    def sidx(ch):
        return (iota + ch * ROWS_PER_CHUNK) * K + wid

    def gather(ch):
        return pltpu.async_copy(x_hbm.at[gidx(ch)], bufs[ch % NBUF],
                                gsems[ch % NBUF])

    gathers = [None] * NUM_CHUNKS
    scatters = [None] * NUM_CHUNKS
    gathers[0] = gather(0)

    for j in range(C // L):
        pos = iota + j * L
        wrow_v[pl.ds(j * L, L)] = jnp.where(pos == c_k, 1.0, 0.0).astype(
            jnp.float32)
    pltpu.sync_copy(wrow_v, w_hbm.at[wid])
    pltpu.sync_copy(wrow_v, w2_hbm.at[wid])

    for ch in range(NUM_CHUNKS):
        gathers[ch].wait()
        scatters[ch] = pltpu.async_copy(
            bufs[ch % NBUF], z_hbm.at[sidx(ch)], ssems[ch % NBUF])
        nxt = ch + 1
        if nxt < NUM_CHUNKS:
            if nxt >= NBUF:
                scatters[nxt - NBUF].wait()
            gathers[nxt] = gather(nxt)
    for ch in range(max(0, NUM_CHUNKS - NBUF), NUM_CHUNKS):
        scatters[ch].wait()


@jax.jit
def _run(x_flat, alpha):
    mesh = plsc.VectorSubcoreMesh(core_axis_name="c", subcore_axis_name="s")
    fn = functools.partial(
        pl.kernel, mesh=mesh,
        compiler_params=pltpu.CompilerParams(needs_layout_passes=False),
        out_type=[
            jax.ShapeDtypeStruct((B * K, T), jnp.float32),
            jax.ShapeDtypeStruct((K, C), jnp.float32),
            jax.ShapeDtypeStruct((K, C), jnp.float32),
        ],
        scratch_types=(
            [pltpu.VMEM((C,), jnp.float32)] * 2
            + [pltpu.VMEM((ROWS_PER_CHUNK, T), jnp.float32)] * NBUF
            + [pltpu.SemaphoreType.DMA] * (2 * NBUF)
        ),
    )(_selector_dup_kernel)
    return fn(x_flat, alpha)


def kernel(x, alpha):
    z_flat, w_hard, w_hard2 = _run(x.reshape(B * C, T), alpha)
    return (z_flat.reshape(B, 1, K, T), w_hard, w_hard2)


# speedup vs baseline: 5.3819x; 1.0532x over previous
import functools

import jax
import jax.numpy as jnp
from jax import lax
from jax.experimental import pallas as pl
from jax.experimental.pallas import tpu as pltpu
from jax.experimental.pallas import tpu_sc as plsc

B, C, T, K = 64, 256, 2048, 32

L = 16
NBUF = 3
ROWS_PER_CHUNK = 16
NUM_CHUNKS = B // ROWS_PER_CHUNK


def _selector_dup_kernel(x_hbm, alpha_hbm, z_hbm, w_hbm, w2_hbm,
                         arow_v, wrow_v,
                         buf0, buf1, buf2,
                         gsem0, gsem1, gsem2,
                         ssem0, ssem1, ssem2):
    nc = 2
    wid = lax.axis_index("s") * nc + lax.axis_index("c")

    pltpu.sync_copy(alpha_hbm.at[wid], arow_v)
    iota = lax.iota(jnp.int32, L)
    best_v = arow_v[pl.ds(0, L)]
    best_i = iota
    for j in range(1, C // L):
        v = arow_v[pl.ds(j * L, L)]
        pos = iota + j * L
        upd = v > best_v
        best_v = jnp.where(upd, v, best_v)
        best_i = jnp.where(upd, pos, best_i)
    sk, _ = plsc.sort_key_val(best_v, best_i)
    m = sk[15]
    cand = jnp.where(best_v == m, best_i, jnp.int32(C))
    ck_sorted, _ = plsc.sort_key_val(cand, cand)
    c_k = ck_sorted[0]

    bufs = (buf0, buf1, buf2)
    gsems = (gsem0, gsem1, gsem2)
    ssems = (ssem0, ssem1, ssem2)

    def gidx(ch):
        return (iota + ch * ROWS_PER_CHUNK) * C + c_k

    def sidx(ch):
        return (iota + ch * ROWS_PER_CHUNK) * K + wid

    def gather(ch):
        return pltpu.async_copy(x_hbm.at[gidx(ch)], bufs[ch % NBUF],
                                gsems[ch % NBUF])

    gathers = [None] * NUM_CHUNKS
    scatters = [None] * NUM_CHUNKS
    for ch in range(NBUF - 1):
        gathers[ch] = gather(ch)

    for j in range(C // L):
        pos = iota + j * L
        wrow_v[pl.ds(j * L, L)] = jnp.where(pos == c_k, 1.0, 0.0).astype(
            jnp.float32)
    pltpu.sync_copy(wrow_v, w_hbm.at[wid])
    pltpu.sync_copy(wrow_v, w2_hbm.at[wid])

    for ch in range(NUM_CHUNKS):
        nxt = ch + NBUF - 1
        if nxt < NUM_CHUNKS:
            if ch >= 1:
                scatters[ch - 1].wait()
            gathers[nxt] = gather(nxt)
        gathers[ch].wait()
        scatters[ch] = pltpu.async_copy(
            bufs[ch % NBUF], z_hbm.at[sidx(ch)], ssems[ch % NBUF])
    for ch in range(max(0, NUM_CHUNKS - NBUF), NUM_CHUNKS):
        scatters[ch].wait()


@jax.jit
def _run(x_flat, alpha):
    mesh = plsc.VectorSubcoreMesh(core_axis_name="c", subcore_axis_name="s")
    fn = functools.partial(
        pl.kernel, mesh=mesh,
        compiler_params=pltpu.CompilerParams(needs_layout_passes=False),
        out_type=[
            jax.ShapeDtypeStruct((B * K, T), jnp.float32),
            jax.ShapeDtypeStruct((K, C), jnp.float32),
            jax.ShapeDtypeStruct((K, C), jnp.float32),
        ],
        scratch_types=(
            [pltpu.VMEM((C,), jnp.float32)] * 2
            + [pltpu.VMEM((ROWS_PER_CHUNK, T), jnp.float32)] * NBUF
            + [pltpu.SemaphoreType.DMA] * (2 * NBUF)
        ),
    )(_selector_dup_kernel)
    return fn(x_flat, alpha)


def kernel(x, alpha):
    z_flat, w_hard, w_hard2 = _run(x.reshape(B * C, T), alpha)
    return (z_flat.reshape(B, 1, K, T), w_hard, w_hard2)
